# baseline (speedup 1.0000x reference)
import functools

import jax
import jax.numpy as jnp
from jax import lax
from jax.experimental import pallas as pl
from jax.experimental.pallas import tpu as pltpu
from jax.experimental.pallas import tpu_sc as plsc

N_NODES = 10000
N_EDGES = 320000
NC = 2
NS = 16
CHUNK = 80
SLAB_ROWS = N_EDGES // NS // CHUNK
BC = 5
BUF_E = BC * CHUNK
NBC_FULL = SLAB_ROWS // BC
NBC_HALF = SLAB_ROWS // NC // BC
ZROWS = 104

_SC_PARAMS = pltpu.CompilerParams(use_tc_tiling_on_sc=False)


def _mesh():
    return plsc.VectorSubcoreMesh(
        core_axis_name="c", subcore_axis_name="s", num_cores=NC, num_subcores=NS
    )


def _zero_vmem(buf, rows, d):
    def zero_row(r, carry):
        for j in range(d // 16):
            buf[r, pl.ds(16 * j, 16)] = jnp.zeros((16,), jnp.float32)
        return carry

    lax.fori_loop(0, rows, zero_row, 0)


def _node_span(s):
    return 624 * s


def _zero_acc(acc, zbuf, s):
    base = _node_span(s)
    for i in range(6):
        pltpu.sync_copy(zbuf, acc.at[pl.ds(base + ZROWS * i, ZROWS)])

    @pl.when(s == 15)
    def _():
        pltpu.sync_copy(zbuf.at[pl.ds(0, 16)], acc.at[pl.ds(9984, 16)])


def _copy_out(acc, bounce, out_slice, s):
    base = _node_span(s)
    for i in range(6):
        sl = pl.ds(base + ZROWS * i, ZROWS)
        pltpu.sync_copy(acc.at[sl], bounce.at[pl.ds(0, ZROWS)])
        pltpu.sync_copy(bounce.at[pl.ds(0, ZROWS)], out_slice.at[sl])

    @pl.when(s == 15)
    def _():
        pltpu.sync_copy(acc.at[pl.ds(9984, 16)], bounce.at[pl.ds(0, 16)])
        pltpu.sync_copy(bounce.at[pl.ds(0, 16)], out_slice.at[pl.ds(9984, 16)])


def _edge_pipeline(src_slab, dst_slab, g_src, acc, isrc, idst, isems, bufs,
                   gsems, ssems, row_lo, n_rows):
    GR = 25
    GE = GR * CHUNK
    NGRP = n_rows // GR
    NBC = n_rows // BC
    BPG = GR // BC
    NB = len(bufs)
    gpend = {b: None for b in range(NB)}
    spend = {b: None for b in range(NB)}
    ipend = {0: None, 1: None}

    def issue_idx(grp):
        st = grp % 2
        sl = pl.ds(row_lo * CHUNK + grp * GE, GE)
        return [
            pltpu.async_copy(src_slab.at[sl], isrc[st], isems[st]),
            pltpu.async_copy(dst_slab.at[sl], idst[st], isems[st]),
        ]

    def issue_gathers(k, b):
        st = (k // BPG) % 2
        lo = (k % BPG) * BUF_E
        return [
            pltpu.async_copy(
                g_src.at[isrc[st].at[pl.ds(lo, BUF_E)]],
                bufs[b],
                gsems[b],
            )
        ]

    def issue_scatters(k, b):
        st = (k // BPG) % 2
        lo = (k % BPG) * BUF_E
        return [
            pltpu.async_copy(
                bufs[b],
                acc.at[idst[st].at[pl.ds(lo, BUF_E)]],
                ssems[b],
                add=True,
            )
        ]

    ipend[0] = issue_idx(0)
    for k in range(NBC):
        b = k % NB
        grp = k // BPG
        if k % BPG == 0 and ipend[grp % 2] is not None:
            for d_ in ipend[grp % 2]:
                d_.wait()
            ipend[grp % 2] = None
        if spend[b] is not None:
            for d_ in spend[b]:
                d_.wait()
            spend[b] = None
        gpend[b] = issue_gathers(k, b)
        if k % BPG == 2 and grp + 1 < NGRP:
            ipend[(grp + 1) % 2] = issue_idx(grp + 1)
        bp = (k - 1) % NB
        if k >= 1 and gpend[bp] is not None:
            for d_ in gpend[bp]:
                d_.wait()
            gpend[bp] = None
            spend[bp] = issue_scatters(k - 1, bp)
    b_last = (NBC - 1) % NB
    for d_ in gpend[b_last]:
        d_.wait()
    spend[b_last] = issue_scatters(NBC - 1, b_last)
    for b in range(NB):
        if spend[b] is not None:
            for d_ in spend[b]:
                d_.wait()


@functools.partial(
    pl.kernel,
    mesh=_mesh(),
    compiler_params=_SC_PARAMS,
    out_type=jax.ShapeDtypeStruct((NC * N_NODES,), jnp.float32),
    scratch_types=[
        pltpu.VMEM((SLAB_ROWS // NC * CHUNK,), jnp.int32),
        pltpu.VMEM((BUF_E,), jnp.float32),
        pltpu.VMEM((ZROWS,), jnp.float32),
        pltpu.VMEM_SHARED((N_NODES,), jnp.float32),
        pltpu.SemaphoreType.DMA,
    ],
)
def _deg_kernel(dst_hbm, out_hbm, dstv, ones, zbuf, acc, dsem):
    c = lax.axis_index("c")
    s = lax.axis_index("s")
    ne = SLAB_ROWS // NC * CHUNK
    pltpu.sync_copy(dst_hbm.at[s, pl.ds(c * ne, ne)], dstv)
    for i in range(BUF_E // 16):
        ones[pl.ds(16 * i, 16)] = jnp.ones((16,), jnp.float32)
    for i in range(6):
        zbuf[pl.ds(16 * i, 16)] = jnp.zeros((16,), jnp.float32)
    zbuf[pl.ds(88, 16)] = jnp.zeros((16,), jnp.float32)
    base = 624 * s
    for i in range(6):
        pltpu.sync_copy(zbuf, acc.at[pl.ds(base + ZROWS * i, ZROWS)])

    @pl.when(s == 15)
    def _():
        pltpu.sync_copy(zbuf.at[pl.ds(0, 16)], acc.at[pl.ds(9984, 16)])

    plsc.subcore_barrier()

    def fire(j, carry):
        pltpu.async_copy(
            ones, acc.at[dstv.at[pl.ds(j * BUF_E, BUF_E)]], dsem, add=True
        )
        return carry

    lax.fori_loop(0, ne // BUF_E, fire, 0)

    def drain(j, carry):
        pltpu.make_async_copy(
            ones, acc.at[dstv.at[pl.ds(0, BUF_E)]], dsem
        ).wait()
        return carry

    lax.fori_loop(0, ne // BUF_E, drain, 0)
    plsc.subcore_barrier()
    for i in range(6):
        sl = pl.ds(base + ZROWS * i, ZROWS)
        pltpu.sync_copy(acc.at[sl], zbuf)
        pltpu.sync_copy(zbuf, out_hbm.at[pl.ds(c * N_NODES + base + ZROWS * i, ZROWS)])

    @pl.when(s == 15)
    def _():
        pltpu.sync_copy(acc.at[pl.ds(9984, 16)], zbuf.at[pl.ds(0, 16)])
        pltpu.sync_copy(zbuf.at[pl.ds(0, 16)], out_hbm.at[pl.ds(c * N_NODES + 9984, 16)])


@functools.partial(
    pl.kernel,
    mesh=_mesh(),
    compiler_params=_SC_PARAMS,
    out_type=jax.ShapeDtypeStruct((NC, N_NODES, 64), jnp.float32),
    scratch_types=[
        pltpu.VMEM((25 * CHUNK,), jnp.int32),
        pltpu.VMEM((25 * CHUNK,), jnp.int32),
        pltpu.VMEM((25 * CHUNK,), jnp.int32),
        pltpu.VMEM((25 * CHUNK,), jnp.int32),
        pltpu.VMEM((BUF_E, 64), jnp.float32),
        pltpu.VMEM((BUF_E, 64), jnp.float32),
        pltpu.VMEM((BUF_E, 64), jnp.float32),
        pltpu.VMEM_SHARED((N_NODES, 64), jnp.float32),
        pltpu.SemaphoreType.DMA,
        pltpu.SemaphoreType.DMA,
        pltpu.SemaphoreType.DMA,
        pltpu.SemaphoreType.DMA,
        pltpu.SemaphoreType.DMA,
        pltpu.SemaphoreType.DMA,
        pltpu.SemaphoreType.DMA,
        pltpu.SemaphoreType.DMA,
    ],
)
def _smooth_full(src_hbm, dst_hbm, g_hbm, out_hbm, isrc0, isrc1, idst0, idst1,
                 buf0, buf1, buf2, acc, is0, is1, gs0, gs1, gs2, ss0, ss1, ss2):
    c = lax.axis_index("c")
    s = lax.axis_index("s")
    zb = buf0.at[pl.ds(0, ZROWS)]
    _zero_vmem(zb, ZROWS, 64)
    _zero_acc(acc, zb, s)
    plsc.subcore_barrier()
    _edge_pipeline(
        src_hbm.at[s], dst_hbm.at[s], g_hbm.at[c], acc,
        (isrc0, isrc1), (idst0, idst1), (is0, is1),
        (buf0, buf1, buf2), (gs0, gs1, gs2), (ss0, ss1, ss2),
        0, SLAB_ROWS,
    )
    plsc.subcore_barrier()
    _copy_out(acc, buf0, out_hbm.at[c], s)


@functools.partial(
    pl.kernel,
    mesh=_mesh(),
    compiler_params=_SC_PARAMS,
    out_type=jax.ShapeDtypeStruct((NC, N_NODES, 64), jnp.float32),
    scratch_types=[
        pltpu.VMEM((25 * CHUNK,), jnp.int32),
        pltpu.VMEM((25 * CHUNK,), jnp.int32),
        pltpu.VMEM((25 * CHUNK,), jnp.int32),
        pltpu.VMEM((25 * CHUNK,), jnp.int32),
        pltpu.VMEM((BUF_E, 64), jnp.float32),
        pltpu.VMEM((BUF_E, 64), jnp.float32),
        pltpu.VMEM((BUF_E, 64), jnp.float32),
        pltpu.VMEM_SHARED((N_NODES, 64), jnp.float32),
        pltpu.SemaphoreType.DMA,
        pltpu.SemaphoreType.DMA,
        pltpu.SemaphoreType.DMA,
        pltpu.SemaphoreType.DMA,
        pltpu.SemaphoreType.DMA,
        pltpu.SemaphoreType.DMA,
        pltpu.SemaphoreType.DMA,
        pltpu.SemaphoreType.DMA,
    ],
)
def _smooth_half(src_hbm, dst_hbm, g_hbm, out_hbm, isrc0, isrc1, idst0, idst1,
                 buf0, buf1, buf2, acc, is0, is1, gs0, gs1, gs2, ss0, ss1, ss2):
    c = lax.axis_index("c")
    s = lax.axis_index("s")
    nch = SLAB_ROWS // NC
    zb = buf0.at[pl.ds(0, ZROWS)]
    _zero_vmem(zb, ZROWS, 64)
    _zero_acc(acc, zb, s)
    plsc.subcore_barrier()
    _edge_pipeline(
        src_hbm.at[s], dst_hbm.at[s], g_hbm, acc,
        (isrc0, isrc1), (idst0, idst1), (is0, is1),
        (buf0, buf1, buf2), (gs0, gs1, gs2), (ss0, ss1, ss2),
        c * nch, nch,
    )
    plsc.subcore_barrier()
    _copy_out(acc, buf0, out_hbm.at[c], s)


def _tc_matmul0(x, w0, b0):
    def body(x_ref, w_ref, b_ref, h_ref):
        h = jnp.dot(x_ref[...], w_ref[...], preferred_element_type=jnp.float32)
        h_ref[...] = h + b_ref[...]

    return pl.pallas_call(
        body,
        out_shape=jax.ShapeDtypeStruct((N_NODES, 128), jnp.float32),
    )(x, w0, b0)


def _tc_layer0(deg, h):
    def body(deg_ref, h_ref, norm_ref, g0_ref):
        dsum = deg_ref[0, :] + deg_ref[1, :] + 1.0
        norm = lax.rsqrt(dsum)
        norm_ref[...] = norm[:, None]
        g0 = h_ref[...] * norm[:, None]
        g0_ref[0] = g0[:, :64]
        g0_ref[1] = g0[:, 64:]

    return pl.pallas_call(
        body,
        out_shape=(
            jax.ShapeDtypeStruct((N_NODES, 1), jnp.float32),
            jax.ShapeDtypeStruct((NC, N_NODES, 64), jnp.float32),
        ),
    )(deg, h)


def _tc_layer1(p, g0, norm, w1, b1):
    def body(p_ref, g0_ref, norm_ref, w_ref, b_ref, g1_ref):
        sm = jnp.concatenate(
            [p_ref[0] + g0_ref[0], p_ref[1] + g0_ref[1]], axis=1
        ) * norm_ref[...]
        h1 = jnp.maximum(sm, 0.0)
        o = jnp.dot(h1, w_ref[...], preferred_element_type=jnp.float32)
        o = o + b_ref[...]
        g1_ref[...] = o * norm_ref[...]

    return pl.pallas_call(
        body,
        out_shape=jax.ShapeDtypeStruct((N_NODES, w1.shape[1]), jnp.float32),
    )(p, g0, norm, w1, b1)


def _tc_final(q, g1, norm):
    def body(q_ref, g1_ref, norm_ref, o_ref):
        o_ref[...] = (q_ref[0] + q_ref[1] + g1_ref[...]) * norm_ref[...]

    return pl.pallas_call(
        body,
        out_shape=jax.ShapeDtypeStruct(g1.shape, jnp.float32),
    )(q, g1, norm)


def kernel(X, edge_index, W0, b0, W1, b1):
    src = edge_index[0].astype(jnp.int32).reshape(NS, SLAB_ROWS * CHUNK)
    dst = edge_index[1].astype(jnp.int32).reshape(NS, SLAB_ROWS * CHUNK)
    deg = _deg_kernel(dst).reshape(NC, N_NODES)
    h = _tc_matmul0(X, W0, b0.reshape(1, -1))
    norm, g0 = _tc_layer0(deg, h)
    p = _smooth_full(src, dst, g0)
    g1 = _tc_layer1(p, g0, norm, W1, b1.reshape(1, -1))
    q = _smooth_half(src, dst, g1)
    return _tc_final(q, g1, norm)


# speedup vs baseline: 1.0071x; 1.0071x over previous
import functools

import jax
import jax.numpy as jnp
from jax import lax
from jax.experimental import pallas as pl
from jax.experimental.pallas import tpu as pltpu
from jax.experimental.pallas import tpu_sc as plsc

N_NODES = 10000
N_EDGES = 320000
NC = 2
NS = 16
CHUNK = 80
SLAB_ROWS = N_EDGES // NS // CHUNK
BC = 5
BUF_E = BC * CHUNK
NBC_FULL = SLAB_ROWS // BC
NBC_HALF = SLAB_ROWS // NC // BC
ZROWS = 104

_SC_PARAMS = pltpu.CompilerParams(use_tc_tiling_on_sc=False)


def _mesh():
    return plsc.VectorSubcoreMesh(
        core_axis_name="c", subcore_axis_name="s", num_cores=NC, num_subcores=NS
    )


def _zero_vmem(buf, rows, d):
    def zero_row(r, carry):
        for j in range(d // 16):
            buf[r, pl.ds(16 * j, 16)] = jnp.zeros((16,), jnp.float32)
        return carry

    lax.fori_loop(0, rows, zero_row, 0)


def _node_span(s):
    return 624 * s


def _zero_acc(acc, zbuf, s):
    base = _node_span(s)
    for i in range(6):
        pltpu.sync_copy(zbuf, acc.at[pl.ds(base + ZROWS * i, ZROWS)])

    @pl.when(s == 15)
    def _():
        pltpu.sync_copy(zbuf.at[pl.ds(0, 16)], acc.at[pl.ds(9984, 16)])


def _copy_out(acc, bounce, out_slice, s):
    base = _node_span(s)
    for i in range(6):
        sl = pl.ds(base + ZROWS * i, ZROWS)
        pltpu.sync_copy(acc.at[sl], bounce.at[pl.ds(0, ZROWS)])
        pltpu.sync_copy(bounce.at[pl.ds(0, ZROWS)], out_slice.at[sl])

    @pl.when(s == 15)
    def _():
        pltpu.sync_copy(acc.at[pl.ds(9984, 16)], bounce.at[pl.ds(0, 16)])
        pltpu.sync_copy(bounce.at[pl.ds(0, 16)], out_slice.at[pl.ds(9984, 16)])


def _edge_pipeline(src_slab, dst_slab, g_src, acc, isrc, idst, isems, bufs,
                   gsems, ssems, row_lo, n_rows):
    GR = 25
    GE = GR * CHUNK
    NGRP = n_rows // GR
    NBC = n_rows // BC
    BPG = GR // BC
    NB = len(bufs)
    gpend = {b: None for b in range(NB)}
    spend = {b: None for b in range(NB)}
    ipend = {0: None, 1: None}

    def issue_idx(grp):
        st = grp % 2
        sl = pl.ds(row_lo * CHUNK + grp * GE, GE)
        return [
            pltpu.async_copy(src_slab.at[sl], isrc[st], isems[st]),
            pltpu.async_copy(dst_slab.at[sl], idst[st], isems[st]),
        ]

    def issue_gathers(k, b):
        st = (k // BPG) % 2
        lo = (k % BPG) * BUF_E
        return [
            pltpu.async_copy(
                g_src.at[isrc[st].at[pl.ds(lo, BUF_E)]],
                bufs[b],
                gsems[b],
            )
        ]

    def issue_scatters(k, b):
        st = (k // BPG) % 2
        lo = (k % BPG) * BUF_E
        return [
            pltpu.async_copy(
                bufs[b],
                acc.at[idst[st].at[pl.ds(lo, BUF_E)]],
                ssems[b],
                add=True,
            )
        ]

    ipend[0] = issue_idx(0)
    for k in range(NBC):
        b = k % NB
        grp = k // BPG
        if k % BPG == 0 and ipend[grp % 2] is not None:
            for d_ in ipend[grp % 2]:
                d_.wait()
            ipend[grp % 2] = None
        if spend[b] is not None:
            for d_ in spend[b]:
                d_.wait()
            spend[b] = None
        gpend[b] = issue_gathers(k, b)
        if k % BPG == 2 and grp + 1 < NGRP:
            ipend[(grp + 1) % 2] = issue_idx(grp + 1)
        bp = (k - 1) % NB
        if k >= 1 and gpend[bp] is not None:
            for d_ in gpend[bp]:
                d_.wait()
            gpend[bp] = None
            spend[bp] = issue_scatters(k - 1, bp)
    b_last = (NBC - 1) % NB
    for d_ in gpend[b_last]:
        d_.wait()
    spend[b_last] = issue_scatters(NBC - 1, b_last)
    for b in range(NB):
        if spend[b] is not None:
            for d_ in spend[b]:
                d_.wait()


@functools.partial(
    pl.kernel,
    mesh=_mesh(),
    compiler_params=_SC_PARAMS,
    out_type=jax.ShapeDtypeStruct((NC * N_NODES,), jnp.float32),
    scratch_types=[
        pltpu.VMEM((SLAB_ROWS // NC * CHUNK,), jnp.int32),
        pltpu.VMEM((BUF_E,), jnp.float32),
        pltpu.VMEM((ZROWS,), jnp.float32),
        pltpu.VMEM_SHARED((N_NODES,), jnp.float32),
        pltpu.SemaphoreType.DMA,
    ],
)
def _deg_kernel(dst_hbm, out_hbm, dstv, ones, zbuf, acc, dsem):
    c = lax.axis_index("c")
    s = lax.axis_index("s")
    ne = SLAB_ROWS // NC * CHUNK
    pltpu.sync_copy(dst_hbm.at[s, pl.ds(c * ne, ne)], dstv)
    for i in range(BUF_E // 16):
        ones[pl.ds(16 * i, 16)] = jnp.ones((16,), jnp.float32)
    for i in range(6):
        zbuf[pl.ds(16 * i, 16)] = jnp.zeros((16,), jnp.float32)
    zbuf[pl.ds(88, 16)] = jnp.zeros((16,), jnp.float32)
    base = 624 * s
    for i in range(6):
        pltpu.sync_copy(zbuf, acc.at[pl.ds(base + ZROWS * i, ZROWS)])

    @pl.when(s == 15)
    def _():
        pltpu.sync_copy(zbuf.at[pl.ds(0, 16)], acc.at[pl.ds(9984, 16)])

    plsc.subcore_barrier()

    def fire(j, carry):
        pltpu.async_copy(
            ones, acc.at[dstv.at[pl.ds(j * BUF_E, BUF_E)]], dsem, add=True
        )
        return carry

    lax.fori_loop(0, ne // BUF_E, fire, 0)

    def drain(j, carry):
        pltpu.make_async_copy(
            ones, acc.at[dstv.at[pl.ds(0, BUF_E)]], dsem
        ).wait()
        return carry

    lax.fori_loop(0, ne // BUF_E, drain, 0)
    plsc.subcore_barrier()
    for i in range(6):
        sl = pl.ds(base + ZROWS * i, ZROWS)
        pltpu.sync_copy(acc.at[sl], zbuf)
        pltpu.sync_copy(zbuf, out_hbm.at[pl.ds(c * N_NODES + base + ZROWS * i, ZROWS)])

    @pl.when(s == 15)
    def _():
        pltpu.sync_copy(acc.at[pl.ds(9984, 16)], zbuf.at[pl.ds(0, 16)])
        pltpu.sync_copy(zbuf.at[pl.ds(0, 16)], out_hbm.at[pl.ds(c * N_NODES + 9984, 16)])


@functools.partial(
    pl.kernel,
    mesh=_mesh(),
    compiler_params=_SC_PARAMS,
    out_type=jax.ShapeDtypeStruct((NC, N_NODES, 64), jnp.float32),
    scratch_types=[
        pltpu.VMEM((25 * CHUNK,), jnp.int32),
        pltpu.VMEM((25 * CHUNK,), jnp.int32),
        pltpu.VMEM((25 * CHUNK,), jnp.int32),
        pltpu.VMEM((25 * CHUNK,), jnp.int32),
        pltpu.VMEM((BUF_E, 64), jnp.float32),
        pltpu.VMEM((BUF_E, 64), jnp.float32),
        pltpu.VMEM((BUF_E, 64), jnp.float32),
        pltpu.VMEM_SHARED((N_NODES, 64), jnp.float32),
        pltpu.SemaphoreType.DMA,
        pltpu.SemaphoreType.DMA,
        pltpu.SemaphoreType.DMA,
        pltpu.SemaphoreType.DMA,
        pltpu.SemaphoreType.DMA,
        pltpu.SemaphoreType.DMA,
        pltpu.SemaphoreType.DMA,
        pltpu.SemaphoreType.DMA,
    ],
)
def _smooth_full(src_hbm, dst_hbm, g_hbm, out_hbm, isrc0, isrc1, idst0, idst1,
                 buf0, buf1, buf2, acc, is0, is1, gs0, gs1, gs2, ss0, ss1, ss2):
    c = lax.axis_index("c")
    s = lax.axis_index("s")
    zb = buf0.at[pl.ds(0, ZROWS)]
    _zero_vmem(zb, ZROWS, 64)
    _zero_acc(acc, zb, s)
    plsc.subcore_barrier()
    _edge_pipeline(
        src_hbm.at[s], dst_hbm.at[s], g_hbm.at[c], acc,
        (isrc0, isrc1), (idst0, idst1), (is0, is1),
        (buf0, buf1, buf2), (gs0, gs1, gs2), (ss0, ss1, ss2),
        0, SLAB_ROWS,
    )
    plsc.subcore_barrier()
    _copy_out(acc, buf0, out_hbm.at[c], s)


@functools.partial(
    pl.kernel,
    mesh=_mesh(),
    compiler_params=_SC_PARAMS,
    out_type=jax.ShapeDtypeStruct((NC, N_NODES, 64), jnp.float32),
    scratch_types=[
        pltpu.VMEM((25 * CHUNK,), jnp.int32),
        pltpu.VMEM((25 * CHUNK,), jnp.int32),
        pltpu.VMEM((25 * CHUNK,), jnp.int32),
        pltpu.VMEM((25 * CHUNK,), jnp.int32),
        pltpu.VMEM((BUF_E, 64), jnp.float32),
        pltpu.VMEM((BUF_E, 64), jnp.float32),
        pltpu.VMEM((BUF_E, 64), jnp.float32),
        pltpu.VMEM_SHARED((N_NODES, 64), jnp.float32),
        pltpu.SemaphoreType.DMA,
        pltpu.SemaphoreType.DMA,
        pltpu.SemaphoreType.DMA,
        pltpu.SemaphoreType.DMA,
        pltpu.SemaphoreType.DMA,
        pltpu.SemaphoreType.DMA,
        pltpu.SemaphoreType.DMA,
        pltpu.SemaphoreType.DMA,
    ],
)
def _smooth_half(src_hbm, dst_hbm, g_hbm, out_hbm, isrc0, isrc1, idst0, idst1,
                 buf0, buf1, buf2, acc, is0, is1, gs0, gs1, gs2, ss0, ss1, ss2):
    c = lax.axis_index("c")
    s = lax.axis_index("s")
    nch = SLAB_ROWS // NC
    zb = buf0.at[pl.ds(0, ZROWS)]
    _zero_vmem(zb, ZROWS, 64)
    _zero_acc(acc, zb, s)
    plsc.subcore_barrier()
    _edge_pipeline(
        src_hbm.at[s], dst_hbm.at[s], g_hbm, acc,
        (isrc0, isrc1), (idst0, idst1), (is0, is1),
        (buf0, buf1, buf2), (gs0, gs1, gs2), (ss0, ss1, ss2),
        c * nch, nch,
    )
    plsc.subcore_barrier()
    _copy_out(acc, buf0, out_hbm.at[c], s)


def _tc_layer0(deg, x, w0, b0):
    def body(deg_ref, x_ref, w_ref, b_ref, norm_ref, g0_ref):
        dsum = deg_ref[0, :] + deg_ref[1, :] + 1.0
        norm = lax.rsqrt(dsum)
        norm_ref[...] = norm[:, None]
        h = jnp.dot(x_ref[...], w_ref[...], preferred_element_type=jnp.float32)
        h = h + b_ref[...]
        g0 = h * norm[:, None]
        g0_ref[0] = g0[:, :64]
        g0_ref[1] = g0[:, 64:]

    return pl.pallas_call(
        body,
        out_shape=(
            jax.ShapeDtypeStruct((N_NODES, 1), jnp.float32),
            jax.ShapeDtypeStruct((NC, N_NODES, 64), jnp.float32),
        ),
    )(deg, x, w0, b0)


def _tc_layer1(p, g0, norm, w1, b1):
    def body(p_ref, g0_ref, norm_ref, w_ref, b_ref, g1_ref):
        sm = jnp.concatenate(
            [p_ref[0] + g0_ref[0], p_ref[1] + g0_ref[1]], axis=1
        ) * norm_ref[...]
        h1 = jnp.maximum(sm, 0.0)
        o = jnp.dot(h1, w_ref[...], preferred_element_type=jnp.float32)
        o = o + b_ref[...]
        g1_ref[...] = o * norm_ref[...]

    return pl.pallas_call(
        body,
        out_shape=jax.ShapeDtypeStruct((N_NODES, w1.shape[1]), jnp.float32),
    )(p, g0, norm, w1, b1)


def _tc_final(q, g1, norm):
    def body(q_ref, g1_ref, norm_ref, o_ref):
        o_ref[...] = (q_ref[0] + q_ref[1] + g1_ref[...]) * norm_ref[...]

    return pl.pallas_call(
        body,
        out_shape=jax.ShapeDtypeStruct(g1.shape, jnp.float32),
    )(q, g1, norm)


def kernel(X, edge_index, W0, b0, W1, b1):
    src = edge_index[0].astype(jnp.int32).reshape(NS, SLAB_ROWS * CHUNK)
    dst = edge_index[1].astype(jnp.int32).reshape(NS, SLAB_ROWS * CHUNK)
    deg = _deg_kernel(dst).reshape(NC, N_NODES)
    norm, g0 = _tc_layer0(deg, X, W0, b0.reshape(1, -1))
    p = _smooth_full(src, dst, g0)
    g1 = _tc_layer1(p, g0, norm, W1, b1.reshape(1, -1))
    q = _smooth_half(src, dst, g1)
    return _tc_final(q, g1, norm)
